# Initial kernel scaffold; baseline (speedup 1.0000x reference)
import jax
import jax.numpy as jnp
from jax.experimental import pallas as pl


def kernel(x, edge_index, edge_w, batch, W_emb, b_emb, pos1_W, pos2_W, lin2_W, lin1_W, lin1_b, bn_gamma, bn_beta):
    raise NotImplementedError("write your pallas kernel here")


# speedup vs baseline: 1.8362x; 1.8362x over previous
import functools

import jax
import jax.numpy as jnp
from jax import lax
from jax.experimental import pallas as pl
from jax.experimental.pallas import tpu as pltpu
from jax.experimental.pallas import tpu_sc as plsc

N = 10000
E = 320000
D = 128
L = 4
G = 16
W = 144
NP = 10240
NC = 2
NS = 16
RPS = NP // NS
EPS = E // NS
C = 200

_f32 = jnp.float32


def _prep_body(x_ref, wemb_ref, bemb_ref, out_ref):
    h0 = jnp.dot(x_ref[...], wemb_ref[...], preferred_element_type=_f32)
    h0 = h0 + bemb_ref[...]
    tail = (lax.broadcasted_iota(jnp.int32, (N, W - D), 1) == 0).astype(_f32)
    out_ref[:, 0:D] = h0
    out_ref[:, D:W] = tail


def _prep(x, W_emb, b_emb):
    return pl.pallas_call(
        _prep_body,
        out_shape=jax.ShapeDtypeStruct((N, W), _f32),
    )(x, W_emb, b_emb.reshape(1, D))


TE = 8000


def _u_body(ew_ref, p1_ref, out_ref):
    u = jnp.dot(ew_ref[...], p1_ref[0], preferred_element_type=_f32)
    out_ref[0, :, 0:D] = jnp.maximum(u, 0.0)
    out_ref[0, :, D:W] = jnp.zeros((TE, W - D), _f32)


def _u_all(edge_w, pos1_W):
    return pl.pallas_call(
        _u_body,
        grid=(L, E // TE),
        in_specs=[
            pl.BlockSpec((TE, 2), lambda l, t: (t, 0)),
            pl.BlockSpec((1, 2, D), lambda l, t: (l, 0, 0)),
        ],
        out_specs=pl.BlockSpec((1, TE, W), lambda l, t: (l, t, 0)),
        out_shape=jax.ShapeDtypeStruct((L, E, W), _f32),
    )(edge_w, pos1_W)


def _edge_body(u_hbm, hext_hbm, src_hbm, dst_hbm, zeros_hbm,
               out_u, out_h, acc, sidx, didx, rows, sem):
    c = lax.axis_index("c")
    s = lax.axis_index("s")

    pltpu.sync_copy(zeros_hbm.at[pl.ds(s * RPS, RPS), :],
                    acc.at[pl.ds(s * RPS, RPS), :])
    plsc.subcore_barrier()

    base0 = s * EPS

    def chunk(k, carry):
        b = base0 + k * C
        pltpu.sync_copy(dst_hbm.at[pl.ds(b, C)], didx)

        @pl.when(c == 0)
        def _():
            pltpu.sync_copy(u_hbm.at[pl.ds(b, C), :], rows)

        @pl.when(c == 1)
        def _():
            pltpu.sync_copy(src_hbm.at[pl.ds(b, C)], sidx)
            pltpu.async_copy(hext_hbm.at[sidx], rows, sem).wait()

        pltpu.sync_copy(rows, acc.at[didx], add=True)
        return carry

    lax.fori_loop(0, EPS // C, chunk, 0)
    plsc.subcore_barrier()

    @pl.when(c == 0)
    def _():
        pltpu.sync_copy(acc.at[pl.ds(s * RPS, RPS), :],
                        out_u.at[pl.ds(s * RPS, RPS), :])

    @pl.when(c == 1)
    def _():
        pltpu.sync_copy(acc.at[pl.ds(s * RPS, RPS), :],
                        out_h.at[pl.ds(s * RPS, RPS), :])


_edge_pass = pl.kernel(
    _edge_body,
    out_type=(jax.ShapeDtypeStruct((NP, W), _f32),
              jax.ShapeDtypeStruct((NP, W), _f32)),
    mesh=plsc.VectorSubcoreMesh(core_axis_name="c", subcore_axis_name="s",
                                num_cores=NC, num_subcores=NS),
    scratch_types=[
        pltpu.VMEM_SHARED((NP, W), _f32),
        pltpu.VMEM((C,), jnp.int32),
        pltpu.VMEM((C,), jnp.int32),
        pltpu.VMEM((C, W), _f32),
        pltpu.SemaphoreType.DMA,
    ],
    compiler_params=pltpu.CompilerParams(use_tc_tiling_on_sc=False),
)


def _update_core(su_ref, sh_ref, p2_ref, l2_ref, l1_ref, b1_ref,
                 gam_ref, bet_ref):
    su = su_ref[0:N, 0:D]
    sh = sh_ref[0:N, 0:D]
    cnt = sh_ref[0:N, D:W][:, 0:1]
    m = jnp.dot(su, p2_ref[0], preferred_element_type=_f32) + sh
    m = jnp.dot(m, l2_ref[0], preferred_element_type=_f32)
    aggr = m / jnp.maximum(cnt, 1.0)
    h2 = jnp.dot(aggr, l1_ref[0], preferred_element_type=_f32) + b1_ref[...]
    mu = jnp.mean(h2, axis=0, keepdims=True)
    var = jnp.mean((h2 - mu) * (h2 - mu), axis=0, keepdims=True)
    hn = (h2 - mu) * lax.rsqrt(var + 1e-5) * gam_ref[...] + bet_ref[...]
    return jnp.maximum(hn, 0.0)


def _layer_body(su_ref, sh_ref, p2_ref, l2_ref, l1_ref, b1_ref,
                gam_ref, bet_ref, out_ref):
    h = _update_core(su_ref, sh_ref, p2_ref, l2_ref, l1_ref, b1_ref,
                     gam_ref, bet_ref)
    tail = (lax.broadcasted_iota(jnp.int32, (N, W - D), 1) == 0).astype(_f32)
    out_ref[:, 0:D] = h
    out_ref[:, D:W] = tail


def _layer_update(su, sh, p2, l2, l1, b1, gam, bet):
    return pl.pallas_call(
        _layer_body,
        out_shape=jax.ShapeDtypeStruct((N, W), _f32),
    )(su, sh, p2, l2, l1, b1, gam, bet)


def _final_body(su_ref, sh_ref, p2_ref, l2_ref, l1_ref, b1_ref,
                gam_ref, bet_ref, batch_ref, out_ref):
    h = _update_core(su_ref, sh_ref, p2_ref, l2_ref, l1_ref, b1_ref,
                     gam_ref, bet_ref)
    bt = jnp.broadcast_to(batch_ref[...], (G, N))
    onehot = (lax.broadcasted_iota(jnp.int32, (G, N), 0) == bt).astype(_f32)
    pool = jnp.dot(onehot, h, preferred_element_type=_f32)
    cnts = jnp.sum(onehot, axis=1, keepdims=True)
    out_ref[...] = pool / jnp.maximum(cnts, 1.0)


def _final_update(su, sh, p2, l2, l1, b1, gam, bet, batch2d):
    return pl.pallas_call(
        _final_body,
        out_shape=jax.ShapeDtypeStruct((G, D), _f32),
    )(su, sh, p2, l2, l1, b1, gam, bet, batch2d)


def kernel(x, edge_index, edge_w, batch, W_emb, b_emb, pos1_W, pos2_W,
           lin2_W, lin1_W, lin1_b, bn_gamma, bn_beta):
    src = edge_index[0]
    dst = edge_index[1]
    zeros = jnp.zeros((NP, W), _f32)

    hext = _prep(x, W_emb, b_emb)
    u_all = _u_all(edge_w, pos1_W)

    for i in range(L):
        su, sh = _edge_pass(u_all[i], hext, src, dst, zeros)
        p2 = pos2_W[i:i + 1]
        l2 = lin2_W[i:i + 1]
        l1 = lin1_W[i:i + 1]
        b1 = lin1_b[i].reshape(1, D)
        gam = bn_gamma[i].reshape(1, D)
        bet = bn_beta[i].reshape(1, D)
        if i < L - 1:
            hext = _layer_update(su, sh, p2, l2, l1, b1, gam, bet)
        else:
            out = _final_update(su, sh, p2, l2, l1, b1, gam, bet,
                                batch.reshape(1, N))
    return out


# speedup vs baseline: 3.5913x; 1.9558x over previous
import functools

import jax
import jax.numpy as jnp
from jax import lax
from jax.experimental import pallas as pl
from jax.experimental.pallas import tpu as pltpu
from jax.experimental.pallas import tpu_sc as plsc

N = 10000
E = 320000
D = 128
L = 4
G = 16
NP = 10240
NC = 2
NS = 16
RPS = NP // NS
EPS = E // NS
EPW = E // (NC * NS)
C = 200
CW = 16

_f32 = jnp.float32


def _prep_body(x_ref, wemb_ref, bemb_ref, out_ref):
    h0 = jnp.dot(x_ref[...], wemb_ref[...], preferred_element_type=_f32)
    out_ref[...] = h0 + bemb_ref[...]


def _prep(x, W_emb, b_emb):
    return pl.pallas_call(
        _prep_body,
        out_shape=jax.ShapeDtypeStruct((N, D), _f32),
    )(x, W_emb, b_emb.reshape(1, D))


TE = 8000


def _u_body(ew_ref, p1_ref, out_ref):
    u = jnp.dot(ew_ref[...], p1_ref[...], preferred_element_type=_f32)
    out_ref[...] = jnp.maximum(u, 0.0)


def _u_layer(edge_w, pos1_Wi):
    return pl.pallas_call(
        _u_body,
        grid=(E // TE,),
        in_specs=[
            pl.BlockSpec((TE, 2), lambda t: (t, 0)),
            pl.BlockSpec((2, D), lambda t: (0, 0)),
        ],
        out_specs=pl.BlockSpec((TE, D), lambda t: (t, 0)),
        out_shape=jax.ShapeDtypeStruct((E, D), _f32),
    )(edge_w, pos1_Wi)


def _count_body(dst_hbm, ones_hbm, zeros_hbm, out0, out1,
                acc, didx, ones):
    c = lax.axis_index("c")
    s = lax.axis_index("s")

    pltpu.sync_copy(zeros_hbm.at[pl.ds(s * RPS, RPS), :],
                    acc.at[pl.ds(s * RPS, RPS), :])
    pltpu.sync_copy(ones_hbm, ones)
    plsc.subcore_barrier()

    base0 = (c * NS + s) * EPW

    def chunk(k, carry):
        b = base0 + k * C
        pltpu.sync_copy(dst_hbm.at[pl.ds(b, C)], didx)
        pltpu.sync_copy(ones, acc.at[didx], add=True)
        return carry

    lax.fori_loop(0, EPW // C, chunk, 0)
    plsc.subcore_barrier()

    @pl.when(c == 0)
    def _():
        pltpu.sync_copy(acc.at[pl.ds(s * RPS, RPS), :],
                        out0.at[pl.ds(s * RPS, RPS), :])

    @pl.when(c == 1)
    def _():
        pltpu.sync_copy(acc.at[pl.ds(s * RPS, RPS), :],
                        out1.at[pl.ds(s * RPS, RPS), :])


_count_pass = pl.kernel(
    _count_body,
    out_type=(jax.ShapeDtypeStruct((NP, CW), _f32),
              jax.ShapeDtypeStruct((NP, CW), _f32)),
    mesh=plsc.VectorSubcoreMesh(core_axis_name="c", subcore_axis_name="s",
                                num_cores=NC, num_subcores=NS),
    scratch_types=[
        pltpu.VMEM_SHARED((NP, CW), _f32),
        pltpu.VMEM((C,), jnp.int32),
        pltpu.VMEM((C, CW), _f32),
    ],
    compiler_params=pltpu.CompilerParams(use_tc_tiling_on_sc=False),
)


def _edge_body(u_hbm, hext_hbm, src_hbm, dst_hbm, zeros_hbm,
               out_u, out_h, acc, sidx, didx, rows, sem):
    c = lax.axis_index("c")
    s = lax.axis_index("s")

    pltpu.sync_copy(zeros_hbm.at[pl.ds(s * RPS, RPS), :],
                    acc.at[pl.ds(s * RPS, RPS), :])
    plsc.subcore_barrier()

    base0 = s * EPS

    def chunk(k, carry):
        b = base0 + k * C
        pltpu.sync_copy(dst_hbm.at[pl.ds(b, C)], didx)

        @pl.when(c == 0)
        def _():
            pltpu.sync_copy(u_hbm.at[pl.ds(b, C), :], rows)

        @pl.when(c == 1)
        def _():
            pltpu.sync_copy(src_hbm.at[pl.ds(b, C)], sidx)
            pltpu.async_copy(hext_hbm.at[sidx], rows, sem).wait()

        pltpu.sync_copy(rows, acc.at[didx], add=True)
        return carry

    lax.fori_loop(0, EPS // C, chunk, 0)
    plsc.subcore_barrier()

    @pl.when(c == 0)
    def _():
        pltpu.sync_copy(acc.at[pl.ds(s * RPS, RPS), :],
                        out_u.at[pl.ds(s * RPS, RPS), :])

    @pl.when(c == 1)
    def _():
        pltpu.sync_copy(acc.at[pl.ds(s * RPS, RPS), :],
                        out_h.at[pl.ds(s * RPS, RPS), :])


_edge_pass = pl.kernel(
    _edge_body,
    out_type=(jax.ShapeDtypeStruct((NP, D), _f32),
              jax.ShapeDtypeStruct((NP, D), _f32)),
    mesh=plsc.VectorSubcoreMesh(core_axis_name="c", subcore_axis_name="s",
                                num_cores=NC, num_subcores=NS),
    scratch_types=[
        pltpu.VMEM_SHARED((NP, D), _f32),
        pltpu.VMEM((C,), jnp.int32),
        pltpu.VMEM((C,), jnp.int32),
        pltpu.VMEM((C, D), _f32),
        pltpu.SemaphoreType.DMA,
    ],
    compiler_params=pltpu.CompilerParams(use_tc_tiling_on_sc=False),
)


def _update_core(su_ref, sh_ref, c0_ref, c1_ref, p2_ref, l2_ref, l1_ref,
                 b1_ref, gam_ref, bet_ref):
    su = su_ref[0:N, :]
    sh = sh_ref[0:N, :]
    cnt = c0_ref[0:N, 0:1] + c1_ref[0:N, 0:1]
    m = jnp.dot(su, p2_ref[...], preferred_element_type=_f32) + sh
    m = jnp.dot(m, l2_ref[...], preferred_element_type=_f32)
    aggr = m / jnp.maximum(cnt, 1.0)
    h2 = jnp.dot(aggr, l1_ref[...], preferred_element_type=_f32) + b1_ref[...]
    mu = jnp.mean(h2, axis=0, keepdims=True)
    var = jnp.mean((h2 - mu) * (h2 - mu), axis=0, keepdims=True)
    hn = (h2 - mu) * lax.rsqrt(var + 1e-5) * gam_ref[...] + bet_ref[...]
    return jnp.maximum(hn, 0.0)


def _layer_body(su_ref, sh_ref, c0_ref, c1_ref, p2_ref, l2_ref, l1_ref,
                b1_ref, gam_ref, bet_ref, out_ref):
    out_ref[...] = _update_core(su_ref, sh_ref, c0_ref, c1_ref, p2_ref,
                                l2_ref, l1_ref, b1_ref, gam_ref, bet_ref)


def _layer_update(su, sh, c0, c1, p2, l2, l1, b1, gam, bet):
    return pl.pallas_call(
        _layer_body,
        out_shape=jax.ShapeDtypeStruct((N, D), _f32),
    )(su, sh, c0, c1, p2, l2, l1, b1, gam, bet)


def _final_body(su_ref, sh_ref, c0_ref, c1_ref, p2_ref, l2_ref, l1_ref,
                b1_ref, gam_ref, bet_ref, batch_ref, out_ref):
    h = _update_core(su_ref, sh_ref, c0_ref, c1_ref, p2_ref, l2_ref,
                     l1_ref, b1_ref, gam_ref, bet_ref)
    bt = jnp.broadcast_to(batch_ref[...], (G, N))
    onehot = (lax.broadcasted_iota(jnp.int32, (G, N), 0) == bt).astype(_f32)
    pool = jnp.dot(onehot, h, preferred_element_type=_f32)
    cnts = jnp.sum(onehot, axis=1, keepdims=True)
    out_ref[...] = pool / jnp.maximum(cnts, 1.0)


def _final_update(su, sh, c0, c1, p2, l2, l1, b1, gam, bet, batch2d):
    return pl.pallas_call(
        _final_body,
        out_shape=jax.ShapeDtypeStruct((G, D), _f32),
    )(su, sh, c0, c1, p2, l2, l1, b1, gam, bet, batch2d)


def kernel(x, edge_index, edge_w, batch, W_emb, b_emb, pos1_W, pos2_W,
           lin2_W, lin1_W, lin1_b, bn_gamma, bn_beta):
    src = edge_index[0]
    dst = edge_index[1]
    zeros = jnp.zeros((NP, D), _f32)
    zeros16 = jnp.zeros((NP, CW), _f32)
    ones16 = jnp.ones((C, CW), _f32)

    hext = _prep(x, W_emb, b_emb)
    c0, c1 = _count_pass(dst, ones16, zeros16)
    u_layers = [_u_layer(edge_w, pos1_W[i]) for i in range(L)]

    for i in range(L):
        su, sh = _edge_pass(u_layers[i], hext, src, dst, zeros)
        p2 = pos2_W[i]
        l2 = lin2_W[i]
        l1 = lin1_W[i]
        b1 = lin1_b[i].reshape(1, D)
        gam = bn_gamma[i].reshape(1, D)
        bet = bn_beta[i].reshape(1, D)
        if i < L - 1:
            hext = _layer_update(su, sh, c0, c1, p2, l2, l1, b1, gam, bet)
        else:
            out = _final_update(su, sh, c0, c1, p2, l2, l1, b1, gam, bet,
                                batch.reshape(1, N))
    return out


# speedup vs baseline: 5.3562x; 1.4914x over previous
import functools

import jax
import jax.numpy as jnp
from jax import lax
from jax.experimental import pallas as pl
from jax.experimental.pallas import tpu as pltpu
from jax.experimental.pallas import tpu_sc as plsc

N = 10000
E = 320000
D = 128
L = 4
G = 16
NP = 10240
NC = 2
NS = 16
RPS = NP // NS
EPS = E // NS
EPW = E // (NC * NS)
C = 160
CC = 200
CW = 16

_f32 = jnp.float32


def _prep_body(x_ref, wemb_ref, bemb_ref, out_ref):
    h0 = jnp.dot(x_ref[...], wemb_ref[...], preferred_element_type=_f32)
    out_ref[...] = h0 + bemb_ref[...]


def _prep(x, W_emb, b_emb):
    return pl.pallas_call(
        _prep_body,
        out_shape=jax.ShapeDtypeStruct((N, D), _f32),
    )(x, W_emb, b_emb.reshape(1, D))


TE = 8000


def _u_body(ew_ref, p1_ref, out_ref):
    u = jnp.dot(ew_ref[...], p1_ref[...], preferred_element_type=_f32)
    out_ref[...] = jnp.maximum(u, 0.0)


def _u_layer(edge_w, pos1_Wi):
    return pl.pallas_call(
        _u_body,
        grid=(E // TE,),
        in_specs=[
            pl.BlockSpec((TE, 2), lambda t: (t, 0)),
            pl.BlockSpec((2, D), lambda t: (0, 0)),
        ],
        out_specs=pl.BlockSpec((TE, D), lambda t: (t, 0)),
        out_shape=jax.ShapeDtypeStruct((E, D), _f32),
    )(edge_w, pos1_Wi)


def _count_body(dst_hbm, ones_hbm, zeros_hbm, out0, out1,
                acc, didx, ones):
    c = lax.axis_index("c")
    s = lax.axis_index("s")

    pltpu.sync_copy(zeros_hbm.at[pl.ds(s * RPS, RPS), :],
                    acc.at[pl.ds(s * RPS, RPS), :])
    pltpu.sync_copy(ones_hbm, ones)
    plsc.subcore_barrier()

    base0 = (c * NS + s) * EPW

    def chunk(k, carry):
        b = base0 + k * CC
        pltpu.sync_copy(dst_hbm.at[pl.ds(b, CC)], didx)
        pltpu.sync_copy(ones, acc.at[didx], add=True)
        return carry

    lax.fori_loop(0, EPW // CC, chunk, 0)
    plsc.subcore_barrier()

    @pl.when(c == 0)
    def _():
        pltpu.sync_copy(acc.at[pl.ds(s * RPS, RPS), :],
                        out0.at[pl.ds(s * RPS, RPS), :])

    @pl.when(c == 1)
    def _():
        pltpu.sync_copy(acc.at[pl.ds(s * RPS, RPS), :],
                        out1.at[pl.ds(s * RPS, RPS), :])


_count_pass = pl.kernel(
    _count_body,
    out_type=(jax.ShapeDtypeStruct((NP, CW), _f32),
              jax.ShapeDtypeStruct((NP, CW), _f32)),
    mesh=plsc.VectorSubcoreMesh(core_axis_name="c", subcore_axis_name="s",
                                num_cores=NC, num_subcores=NS),
    scratch_types=[
        pltpu.VMEM_SHARED((NP, CW), _f32),
        pltpu.VMEM((CC,), jnp.int32),
        pltpu.VMEM((CC, CW), _f32),
    ],
    compiler_params=pltpu.CompilerParams(use_tc_tiling_on_sc=False),
)


NCH = EPS // C
NCHH = NCH // 2


def _edge_body(u_hbm, hext_hbm, src_hbm, dst_hbm, zeros_hbm,
               out_u, out_h, acc, sidxA, sidxB, didxA, didxB,
               rowsA, rowsB, ia, ib, ra, rb):
    c = lax.axis_index("c")
    s = lax.axis_index("s")

    pltpu.sync_copy(zeros_hbm.at[pl.ds(s * RPS, RPS), :],
                    acc.at[pl.ds(s * RPS, RPS), :])
    plsc.subcore_barrier()

    base0 = s * EPS

    def load_idx(k, didx, sidx, sem):
        b = base0 + k * C
        pltpu.async_copy(dst_hbm.at[pl.ds(b, C)], didx, sem)

        @pl.when(c == 1)
        def _():
            pltpu.async_copy(src_hbm.at[pl.ds(b, C)], sidx, sem)

    def drain_idx(didx, sidx, sem):
        pltpu.make_async_copy(dst_hbm.at[pl.ds(0, C)], didx, sem).wait()

        @pl.when(c == 1)
        def _():
            pltpu.make_async_copy(src_hbm.at[pl.ds(0, C)], sidx, sem).wait()

    def load_rows(k, sidx, buf, sem):
        @pl.when(c == 0)
        def _():
            pltpu.async_copy(u_hbm.at[pl.ds(base0 + k * C, C), :], buf, sem)

        @pl.when(c == 1)
        def _():
            pltpu.async_copy(hext_hbm.at[sidx], buf, sem)

    def drain_rows(buf, sem):
        pltpu.make_async_copy(u_hbm.at[pl.ds(0, C), :], buf, sem).wait()

    load_idx(0, didxA, sidxA, ia)
    load_idx(1, didxB, sidxB, ib)
    drain_idx(didxA, sidxA, ia)
    load_rows(0, sidxA, rowsA, ra)

    def body(g, carry):
        kA = 2 * g
        kB = kA + 1
        drain_idx(didxB, sidxB, ib)
        load_rows(kB, sidxB, rowsB, rb)
        drain_rows(rowsA, ra)
        pltpu.sync_copy(rowsA, acc.at[didxA], add=True)

        @pl.when(kA + 2 < NCH)
        def _():
            load_idx(kA + 2, didxA, sidxA, ia)
            drain_idx(didxA, sidxA, ia)
            load_rows(kA + 2, sidxA, rowsA, ra)

        drain_rows(rowsB, rb)
        pltpu.sync_copy(rowsB, acc.at[didxB], add=True)

        @pl.when(kB + 2 < NCH)
        def _():
            load_idx(kB + 2, didxB, sidxB, ib)

        return carry

    lax.fori_loop(0, NCHH, body, 0)

    drain_rows(rowsA, ra)
    pltpu.sync_copy(rowsA, acc.at[didxA], add=True)
    plsc.subcore_barrier()

    @pl.when(c == 0)
    def _():
        pltpu.sync_copy(acc.at[pl.ds(s * RPS, RPS), :],
                        out_u.at[pl.ds(s * RPS, RPS), :])

    @pl.when(c == 1)
    def _():
        pltpu.sync_copy(acc.at[pl.ds(s * RPS, RPS), :],
                        out_h.at[pl.ds(s * RPS, RPS), :])


_edge_pass = pl.kernel(
    _edge_body,
    out_type=(jax.ShapeDtypeStruct((NP, D), _f32),
              jax.ShapeDtypeStruct((NP, D), _f32)),
    mesh=plsc.VectorSubcoreMesh(core_axis_name="c", subcore_axis_name="s",
                                num_cores=NC, num_subcores=NS),
    scratch_types=[
        pltpu.VMEM_SHARED((NP, D), _f32),
        pltpu.VMEM((C,), jnp.int32),
        pltpu.VMEM((C,), jnp.int32),
        pltpu.VMEM((C,), jnp.int32),
        pltpu.VMEM((C,), jnp.int32),
        pltpu.VMEM((C, D), _f32),
        pltpu.VMEM((C, D), _f32),
        pltpu.SemaphoreType.DMA,
        pltpu.SemaphoreType.DMA,
        pltpu.SemaphoreType.DMA,
        pltpu.SemaphoreType.DMA,
    ],
    compiler_params=pltpu.CompilerParams(use_tc_tiling_on_sc=False),
)


def _update_core(su_ref, sh_ref, c0_ref, c1_ref, p2_ref, l2_ref, l1_ref,
                 b1_ref, gam_ref, bet_ref):
    su = su_ref[0:N, :]
    sh = sh_ref[0:N, :]
    cnt = c0_ref[0:N, 0:1] + c1_ref[0:N, 0:1]
    m = jnp.dot(su, p2_ref[...], preferred_element_type=_f32) + sh
    m = jnp.dot(m, l2_ref[...], preferred_element_type=_f32)
    aggr = m / jnp.maximum(cnt, 1.0)
    h2 = jnp.dot(aggr, l1_ref[...], preferred_element_type=_f32) + b1_ref[...]
    mu = jnp.mean(h2, axis=0, keepdims=True)
    var = jnp.mean((h2 - mu) * (h2 - mu), axis=0, keepdims=True)
    hn = (h2 - mu) * lax.rsqrt(var + 1e-5) * gam_ref[...] + bet_ref[...]
    return jnp.maximum(hn, 0.0)


def _layer_body(su_ref, sh_ref, c0_ref, c1_ref, p2_ref, l2_ref, l1_ref,
                b1_ref, gam_ref, bet_ref, out_ref):
    out_ref[...] = _update_core(su_ref, sh_ref, c0_ref, c1_ref, p2_ref,
                                l2_ref, l1_ref, b1_ref, gam_ref, bet_ref)


def _layer_update(su, sh, c0, c1, p2, l2, l1, b1, gam, bet):
    return pl.pallas_call(
        _layer_body,
        out_shape=jax.ShapeDtypeStruct((N, D), _f32),
    )(su, sh, c0, c1, p2, l2, l1, b1, gam, bet)


def _final_body(su_ref, sh_ref, c0_ref, c1_ref, p2_ref, l2_ref, l1_ref,
                b1_ref, gam_ref, bet_ref, batch_ref, out_ref):
    h = _update_core(su_ref, sh_ref, c0_ref, c1_ref, p2_ref, l2_ref,
                     l1_ref, b1_ref, gam_ref, bet_ref)
    bt = jnp.broadcast_to(batch_ref[...], (G, N))
    onehot = (lax.broadcasted_iota(jnp.int32, (G, N), 0) == bt).astype(_f32)
    pool = jnp.dot(onehot, h, preferred_element_type=_f32)
    cnts = jnp.sum(onehot, axis=1, keepdims=True)
    out_ref[...] = pool / jnp.maximum(cnts, 1.0)


def _final_update(su, sh, c0, c1, p2, l2, l1, b1, gam, bet, batch2d):
    return pl.pallas_call(
        _final_body,
        out_shape=jax.ShapeDtypeStruct((G, D), _f32),
    )(su, sh, c0, c1, p2, l2, l1, b1, gam, bet, batch2d)


def kernel(x, edge_index, edge_w, batch, W_emb, b_emb, pos1_W, pos2_W,
           lin2_W, lin1_W, lin1_b, bn_gamma, bn_beta):
    src = edge_index[0]
    dst = edge_index[1]
    zeros = jnp.zeros((NP, D), _f32)
    zeros16 = jnp.zeros((NP, CW), _f32)
    ones16 = jnp.ones((CC, CW), _f32)

    hext = _prep(x, W_emb, b_emb)
    c0, c1 = _count_pass(dst, ones16, zeros16)
    u_layers = [_u_layer(edge_w, pos1_W[i]) for i in range(L)]

    for i in range(L):
        su, sh = _edge_pass(u_layers[i], hext, src, dst, zeros)
        p2 = pos2_W[i]
        l2 = lin2_W[i]
        l1 = lin1_W[i]
        b1 = lin1_b[i].reshape(1, D)
        gam = bn_gamma[i].reshape(1, D)
        bet = bn_beta[i].reshape(1, D)
        if i < L - 1:
            hext = _layer_update(su, sh, c0, c1, p2, l2, l1, b1, gam, bet)
        else:
            out = _final_update(su, sh, c0, c1, p2, l2, l1, b1, gam, bet,
                                batch.reshape(1, N))
    return out


# speedup vs baseline: 5.6111x; 1.0476x over previous
import functools

import jax
import jax.numpy as jnp
from jax import lax
from jax.experimental import pallas as pl
from jax.experimental.pallas import tpu as pltpu
from jax.experimental.pallas import tpu_sc as plsc

N = 10000
E = 320000
D = 128
L = 4
G = 16
NP = 10240
NC = 2
NS = 16
RPS = NP // NS
EPS = E // NS
EPW = E // (NC * NS)
C = 160
CC = 200
CW = 16

_f32 = jnp.float32


def _prep_body(x_ref, wemb_ref, bemb_ref, out_ref):
    h0 = jnp.dot(x_ref[...], wemb_ref[...], preferred_element_type=_f32)
    out_ref[...] = h0 + bemb_ref[...]


def _prep(x, W_emb, b_emb):
    return pl.pallas_call(
        _prep_body,
        out_shape=jax.ShapeDtypeStruct((N, D), _f32),
    )(x, W_emb, b_emb.reshape(1, D))


TE = 8000


def _u_body(ew_ref, p1_ref, out_ref):
    u = jnp.dot(ew_ref[...], p1_ref[...], preferred_element_type=_f32)
    out_ref[...] = jnp.maximum(u, 0.0)


def _u_layer(edge_w, pos1_Wi):
    return pl.pallas_call(
        _u_body,
        grid=(E // TE,),
        in_specs=[
            pl.BlockSpec((TE, 2), lambda t: (t, 0)),
            pl.BlockSpec((2, D), lambda t: (0, 0)),
        ],
        out_specs=pl.BlockSpec((TE, D), lambda t: (t, 0)),
        out_shape=jax.ShapeDtypeStruct((E, D), _f32),
    )(edge_w, pos1_Wi)


def _count_body(dst_hbm, ones_hbm, zeros_hbm, out0, out1,
                acc, didx, ones):
    c = lax.axis_index("c")
    s = lax.axis_index("s")

    pltpu.sync_copy(zeros_hbm.at[pl.ds(s * RPS, RPS), :],
                    acc.at[pl.ds(s * RPS, RPS), :])
    pltpu.sync_copy(ones_hbm, ones)
    plsc.subcore_barrier()

    base0 = (c * NS + s) * EPW

    def chunk(k, carry):
        b = base0 + k * CC
        pltpu.sync_copy(dst_hbm.at[pl.ds(b, CC)], didx)
        pltpu.sync_copy(ones, acc.at[didx], add=True)
        return carry

    lax.fori_loop(0, EPW // CC, chunk, 0)
    plsc.subcore_barrier()

    @pl.when(c == 0)
    def _():
        pltpu.sync_copy(acc.at[pl.ds(s * RPS, RPS), :],
                        out0.at[pl.ds(s * RPS, RPS), :])

    @pl.when(c == 1)
    def _():
        pltpu.sync_copy(acc.at[pl.ds(s * RPS, RPS), :],
                        out1.at[pl.ds(s * RPS, RPS), :])


_count_pass = pl.kernel(
    _count_body,
    out_type=(jax.ShapeDtypeStruct((NP, CW), _f32),
              jax.ShapeDtypeStruct((NP, CW), _f32)),
    mesh=plsc.VectorSubcoreMesh(core_axis_name="c", subcore_axis_name="s",
                                num_cores=NC, num_subcores=NS),
    scratch_types=[
        pltpu.VMEM_SHARED((NP, CW), _f32),
        pltpu.VMEM((CC,), jnp.int32),
        pltpu.VMEM((CC, CW), _f32),
    ],
    compiler_params=pltpu.CompilerParams(use_tc_tiling_on_sc=False),
)


NCH = EPS // C
NCHQ = NCH // 4


def _edge_body(u_hbm, hext_hbm, src_hbm, dst_hbm, zeros_hbm,
               out_u, out_h, acc,
               sidx0, sidx1, sidx2, sidx3, didx0, didx1, didx2, didx3,
               rowsA, rowsB, i0, i1, i2, i3, ra, rb):
    c = lax.axis_index("c")
    s = lax.axis_index("s")

    pltpu.sync_copy(zeros_hbm.at[pl.ds(s * RPS, RPS), :],
                    acc.at[pl.ds(s * RPS, RPS), :])
    plsc.subcore_barrier()

    base0 = s * EPS
    sl = [(didx0, sidx0, i0), (didx1, sidx1, i1),
          (didx2, sidx2, i2), (didx3, sidx3, i3)]

    def load_idx(k, j):
        didx, sidx, sem = sl[j]
        b = base0 + k * C
        pltpu.async_copy(dst_hbm.at[pl.ds(b, C)], didx, sem)

        @pl.when(c == 1)
        def _():
            pltpu.async_copy(src_hbm.at[pl.ds(b, C)], sidx, sem)

    def drain_idx(j):
        didx, sidx, sem = sl[j]
        pltpu.make_async_copy(dst_hbm.at[pl.ds(0, C)], didx, sem).wait()

        @pl.when(c == 1)
        def _():
            pltpu.make_async_copy(src_hbm.at[pl.ds(0, C)], sidx, sem).wait()

    def load_rows(k, j, buf, sem):
        @pl.when(c == 0)
        def _():
            pltpu.async_copy(u_hbm.at[pl.ds(base0 + k * C, C), :], buf, sem)

        @pl.when(c == 1)
        def _():
            pltpu.async_copy(hext_hbm.at[sl[j][1]], buf, sem)

    def drain_rows(buf, sem):
        pltpu.make_async_copy(u_hbm.at[pl.ds(0, C), :], buf, sem).wait()

    def scatter(buf, j):
        pltpu.sync_copy(buf, acc.at[sl[j][0]], add=True)

    for j in range(4):
        load_idx(j, j)
    drain_idx(0)
    load_rows(0, 0, rowsA, ra)

    def body(g, carry):
        q = 4 * g
        drain_idx(1)
        load_rows(q + 1, 1, rowsB, rb)
        drain_rows(rowsA, ra)
        scatter(rowsA, 0)

        @pl.when(q + 4 < NCH)
        def _():
            load_idx(q + 4, 0)

        drain_idx(2)
        load_rows(q + 2, 2, rowsA, ra)
        drain_rows(rowsB, rb)
        scatter(rowsB, 1)

        @pl.when(q + 5 < NCH)
        def _():
            load_idx(q + 5, 1)

        drain_idx(3)
        load_rows(q + 3, 3, rowsB, rb)
        drain_rows(rowsA, ra)
        scatter(rowsA, 2)

        @pl.when(q + 6 < NCH)
        def _():
            load_idx(q + 6, 2)

        @pl.when(q + 4 < NCH)
        def _():
            drain_idx(0)
            load_rows(q + 4, 0, rowsA, ra)

        drain_rows(rowsB, rb)
        scatter(rowsB, 3)

        @pl.when(q + 7 < NCH)
        def _():
            load_idx(q + 7, 3)

        return carry

    lax.fori_loop(0, NCHQ, body, 0)

    drain_rows(rowsA, ra)
    scatter(rowsA, 0)
    plsc.subcore_barrier()

    @pl.when(c == 0)
    def _():
        pltpu.sync_copy(acc.at[pl.ds(s * RPS, RPS), :],
                        out_u.at[pl.ds(s * RPS, RPS), :])

    @pl.when(c == 1)
    def _():
        pltpu.sync_copy(acc.at[pl.ds(s * RPS, RPS), :],
                        out_h.at[pl.ds(s * RPS, RPS), :])


_edge_pass = pl.kernel(
    _edge_body,
    out_type=(jax.ShapeDtypeStruct((NP, D), _f32),
              jax.ShapeDtypeStruct((NP, D), _f32)),
    mesh=plsc.VectorSubcoreMesh(core_axis_name="c", subcore_axis_name="s",
                                num_cores=NC, num_subcores=NS),
    scratch_types=(
        [pltpu.VMEM_SHARED((NP, D), _f32)]
        + [pltpu.VMEM((C,), jnp.int32) for _ in range(8)]
        + [pltpu.VMEM((C, D), _f32) for _ in range(2)]
        + [pltpu.SemaphoreType.DMA for _ in range(6)]
    ),
    compiler_params=pltpu.CompilerParams(use_tc_tiling_on_sc=False),
)


def _update_core(su_ref, sh_ref, c0_ref, c1_ref, p2_ref, l2_ref, l1_ref,
                 b1_ref, gam_ref, bet_ref):
    su = su_ref[0:N, :]
    sh = sh_ref[0:N, :]
    cnt = c0_ref[0:N, 0:1] + c1_ref[0:N, 0:1]
    m = jnp.dot(su, p2_ref[...], preferred_element_type=_f32) + sh
    m = jnp.dot(m, l2_ref[...], preferred_element_type=_f32)
    aggr = m / jnp.maximum(cnt, 1.0)
    h2 = jnp.dot(aggr, l1_ref[...], preferred_element_type=_f32) + b1_ref[...]
    mu = jnp.mean(h2, axis=0, keepdims=True)
    var = jnp.mean((h2 - mu) * (h2 - mu), axis=0, keepdims=True)
    hn = (h2 - mu) * lax.rsqrt(var + 1e-5) * gam_ref[...] + bet_ref[...]
    return jnp.maximum(hn, 0.0)


def _layer_body(su_ref, sh_ref, c0_ref, c1_ref, p2_ref, l2_ref, l1_ref,
                b1_ref, gam_ref, bet_ref, out_ref):
    out_ref[...] = _update_core(su_ref, sh_ref, c0_ref, c1_ref, p2_ref,
                                l2_ref, l1_ref, b1_ref, gam_ref, bet_ref)


def _layer_update(su, sh, c0, c1, p2, l2, l1, b1, gam, bet):
    return pl.pallas_call(
        _layer_body,
        out_shape=jax.ShapeDtypeStruct((N, D), _f32),
    )(su, sh, c0, c1, p2, l2, l1, b1, gam, bet)


def _final_body(su_ref, sh_ref, c0_ref, c1_ref, p2_ref, l2_ref, l1_ref,
                b1_ref, gam_ref, bet_ref, batch_ref, out_ref):
    h = _update_core(su_ref, sh_ref, c0_ref, c1_ref, p2_ref, l2_ref,
                     l1_ref, b1_ref, gam_ref, bet_ref)
    bt = jnp.broadcast_to(batch_ref[...], (G, N))
    onehot = (lax.broadcasted_iota(jnp.int32, (G, N), 0) == bt).astype(_f32)
    pool = jnp.dot(onehot, h, preferred_element_type=_f32)
    cnts = jnp.sum(onehot, axis=1, keepdims=True)
    out_ref[...] = pool / jnp.maximum(cnts, 1.0)


def _final_update(su, sh, c0, c1, p2, l2, l1, b1, gam, bet, batch2d):
    return pl.pallas_call(
        _final_body,
        out_shape=jax.ShapeDtypeStruct((G, D), _f32),
    )(su, sh, c0, c1, p2, l2, l1, b1, gam, bet, batch2d)


def kernel(x, edge_index, edge_w, batch, W_emb, b_emb, pos1_W, pos2_W,
           lin2_W, lin1_W, lin1_b, bn_gamma, bn_beta):
    src = edge_index[0]
    dst = edge_index[1]
    zeros = jnp.zeros((NP, D), _f32)
    zeros16 = jnp.zeros((NP, CW), _f32)
    ones16 = jnp.ones((CC, CW), _f32)

    hext = _prep(x, W_emb, b_emb)
    c0, c1 = _count_pass(dst, ones16, zeros16)
    u_layers = [_u_layer(edge_w, pos1_W[i]) for i in range(L)]

    for i in range(L):
        su, sh = _edge_pass(u_layers[i], hext, src, dst, zeros)
        p2 = pos2_W[i]
        l2 = lin2_W[i]
        l1 = lin1_W[i]
        b1 = lin1_b[i].reshape(1, D)
        gam = bn_gamma[i].reshape(1, D)
        bet = bn_beta[i].reshape(1, D)
        if i < L - 1:
            hext = _layer_update(su, sh, c0, c1, p2, l2, l1, b1, gam, bet)
        else:
            out = _final_update(su, sh, c0, c1, p2, l2, l1, b1, gam, bet,
                                batch.reshape(1, N))
    return out
